# baseline (speedup 1.0000x reference)
import functools
import jax
import jax.numpy as jnp
from jax import lax
from jax.experimental import pallas as pl
from jax.experimental.pallas import tpu as pltpu

_FRAC = 0.36787944117144233


def _gate_body(k_const, x_ref, o_ref, bits_ref, h16_ref):
    kk = jnp.float32(k_const)
    bits = lax.bitcast_convert_type(x_ref[...], jnp.int32) & jnp.int32(0x7FFFFFFF)
    bits_ref[...] = bits
    h16_ref[...] = lax.shift_right_logical(bits, 16).astype(jnp.int16)

    ones_l = jnp.ones((1, 128), jnp.bfloat16)
    dn = (((1,), (0,)), ((), ()))

    def colsum(m):
        return lax.dot_general(ones_l, m, dn, preferred_element_type=jnp.float32)

    def cnt16(c16):
        m = jnp.where(h16_ref[...] >= c16, jnp.bfloat16(1), jnp.bfloat16(0))
        return jnp.sum(colsum(m))

    def stp1(i, p):
        t = p | (jnp.int32(1) << (jnp.int32(14) - i))
        cnt = cnt16(t.astype(jnp.int16))
        return lax.select(cnt >= kk, t, p)

    p_hi = lax.fori_loop(0, 15, stp1, jnp.int32(0))

    hi = lax.shift_right_logical(bits_ref[...], 16)
    low_q = lax.shift_left(bits_ref[...] & jnp.int32(0xFFFF), 15)
    bits_ref[...] = jnp.where(
        hi == p_hi, low_q, jnp.where(hi > p_hi, jnp.int32(0x7FFFFFFF), jnp.int32(0))
    )
    h16_ref[...] = lax.shift_right_logical(bits_ref[...], 16).astype(jnp.int16)

    def stp2(i, p):
        t = p | (jnp.int32(1) << (jnp.int32(30) - i))
        cnt = cnt16(lax.shift_right_logical(t, 16).astype(jnp.int16))
        return lax.select(cnt >= kk, t, p)

    p_q = lax.fori_loop(0, 15, stp2, jnp.int32(0))

    t = p_q | (jnp.int32(1) << 15)
    mi = jnp.where(bits_ref[...] >= t, jnp.int32(1), jnp.int32(0))
    cnt_i = jnp.sum(jnp.sum(mi, axis=0))
    p_q = lax.select(cnt_i >= jnp.int32(k_const), t, p_q)

    p_full = lax.shift_left(p_hi, 16) | lax.shift_right_logical(p_q, 15)
    abs_bits = lax.bitcast_convert_type(x_ref[...], jnp.int32) & jnp.int32(0x7FFFFFFF)
    o_ref[...] = jnp.where(abs_bits >= p_full, x_ref[...], jnp.float32(0.0))


def kernel(x):
    n = x.size
    k = max(1, int(n * _FRAC))
    if k >= n:
        return x
    return pl.pallas_call(
        functools.partial(_gate_body, k),
        out_shape=jax.ShapeDtypeStruct(x.shape, x.dtype),
        scratch_shapes=[
            pltpu.VMEM(x.shape, jnp.int32),
            pltpu.VMEM(x.shape, jnp.int16),
        ],
    )(x)


# speedup vs baseline: 1.2300x; 1.2300x over previous
import functools
import jax
import jax.numpy as jnp
from jax import lax
from jax.experimental import pallas as pl
from jax.experimental.pallas import tpu as pltpu

_FRAC = 0.36787944117144233


def _gate_body(k_const, x_ref, o_ref, bits_ref, h16_ref):
    kk = jnp.int32(k_const)
    bits = lax.bitcast_convert_type(x_ref[...], jnp.int32) & jnp.int32(0x7FFFFFFF)
    bits_ref[...] = bits
    h16_ref[...] = lax.shift_right_logical(bits, 16).astype(jnp.int16)

    def cnt16(c16):
        cnt = jnp.int32(0)
        for cs in range(4):
            s = jnp.zeros((8, 2048), jnp.int16)
            for blk in range(16):
                s = s + jnp.where(
                    h16_ref[blk * 8 : blk * 8 + 8, cs * 2048 : cs * 2048 + 2048]
                    >= c16,
                    jnp.int16(1),
                    jnp.int16(0),
                )
            s32 = s.astype(jnp.int32)
            cnt = cnt + jnp.sum(jnp.sum(s32, axis=0))
        return cnt

    def stp1(i, p):
        t = p | (jnp.int32(1) << (jnp.int32(14) - i))
        cnt = cnt16(t.astype(jnp.int16))
        return lax.select(cnt >= kk, t, p)

    p_hi = lax.fori_loop(0, 15, stp1, jnp.int32(0))

    hi = lax.shift_right_logical(bits_ref[...], 16)
    low_q = lax.shift_left(bits_ref[...] & jnp.int32(0xFFFF), 15)
    bits_ref[...] = jnp.where(
        hi == p_hi, low_q, jnp.where(hi > p_hi, jnp.int32(0x7FFFFFFF), jnp.int32(0))
    )
    h16_ref[...] = lax.shift_right_logical(bits_ref[...], 16).astype(jnp.int16)

    def stp2(i, p):
        t = p | (jnp.int32(1) << (jnp.int32(30) - i))
        cnt = cnt16(lax.shift_right_logical(t, 16).astype(jnp.int16))
        return lax.select(cnt >= kk, t, p)

    p_q = lax.fori_loop(0, 15, stp2, jnp.int32(0))

    t = p_q | (jnp.int32(1) << 15)
    m = jnp.where(bits_ref[...] >= t, jnp.int32(1), jnp.int32(0))
    cnt = jnp.sum(jnp.sum(m, axis=0))
    p_q = lax.select(cnt >= kk, t, p_q)

    p_full = lax.shift_left(p_hi, 16) | lax.shift_right_logical(p_q, 15)
    abs_bits = lax.bitcast_convert_type(x_ref[...], jnp.int32) & jnp.int32(0x7FFFFFFF)
    o_ref[...] = jnp.where(abs_bits >= p_full, x_ref[...], jnp.float32(0.0))


def kernel(x):
    n = x.size
    k = max(1, int(n * _FRAC))
    if k >= n:
        return x
    return pl.pallas_call(
        functools.partial(_gate_body, k),
        out_shape=jax.ShapeDtypeStruct(x.shape, x.dtype),
        scratch_shapes=[
            pltpu.VMEM(x.shape, jnp.int32),
            pltpu.VMEM(x.shape, jnp.int16),
        ],
    )(x)
